# Initial kernel scaffold; baseline (speedup 1.0000x reference)
import functools

import jax
import jax.numpy as jnp
from jax import lax
from jax.experimental import pallas as pl
from jax.experimental.pallas import tpu as pltpu
from jax.experimental.pallas import tpu_sc as plsc

N = 50000
NACC = 50048
ROWS_PT = NACC // 16
E = 800000
EPAD = 819200
NG = EPAD // 128
TRASH = N
GPW_NARROW = NG // 32
GPW_WIDE = NG // 16
KD = 8
KDW = 4
BLK = 2176
GRID = NACC // BLK

_MESH = plsc.VectorSubcoreMesh(core_axis_name="c", subcore_axis_name="s")


def _zero_acc(zeros_hbm, acc, s):
    r0 = s * ROWS_PT
    pltpu.sync_copy(zeros_hbm.at[pl.ds(r0, ROWS_PT)], acc.at[pl.ds(r0, ROWS_PT)])


def _flush_acc(acc, out_hbm, c, s):
    r0 = s * ROWS_PT
    pltpu.sync_copy(acc.at[pl.ds(r0, ROWS_PT)], out_hbm.at[c, pl.ds(r0, ROWS_PT)])


def _sc_degree_body(dst_hbm, ones_hbm, zeros_hbm, out_hbm, dst_v, ones_v, acc):
    c = lax.axis_index("c")
    s = lax.axis_index("s")
    wid = c * 16 + s
    _zero_acc(zeros_hbm, acc, s)
    pltpu.sync_copy(ones_hbm, ones_v)
    pltpu.sync_copy(dst_hbm.at[pl.ds(wid * GPW_NARROW, GPW_NARROW)], dst_v)
    plsc.subcore_barrier()

    def body(g, carry):
        pltpu.sync_copy(ones_v, acc.at[dst_v.at[g]], add=True)
        return carry

    lax.fori_loop(0, GPW_NARROW, body, 0)
    plsc.subcore_barrier()
    _flush_acc(acc, out_hbm, c, s)


@functools.partial(
    pl.kernel,
    out_type=jax.ShapeDtypeStruct((2, NACC, 1), jnp.float32),
    mesh=_MESH,
    scratch_types=[
        pltpu.VMEM((GPW_NARROW, 128), jnp.int32),
        pltpu.VMEM((128, 1), jnp.float32),
        pltpu.VMEM_SHARED((NACC, 1), jnp.float32),
    ],
)
def _sc_degree(dst_hbm, ones_hbm, zeros_hbm, out_hbm, dst_v, ones_v, acc):
    _sc_degree_body(dst_hbm, ones_hbm, zeros_hbm, out_hbm, dst_v, ones_v, acc)


def _narrow_body(src_hbm, dst_hbm, table_hbm, zeros_hbm, out_hbm,
                 src_v, dst_v, msg_v, acc, sems):
    c = lax.axis_index("c")
    s = lax.axis_index("s")
    wid = c * 16 + s
    g0 = wid * GPW_NARROW
    _zero_acc(zeros_hbm, acc, s)
    pltpu.sync_copy(src_hbm.at[pl.ds(g0, GPW_NARROW)], src_v)
    pltpu.sync_copy(dst_hbm.at[pl.ds(g0, GPW_NARROW)], dst_v)
    plsc.subcore_barrier()

    def chunk(cix, carry):
        base = cix * KD
        descs = [
            pltpu.async_copy(table_hbm.at[src_v.at[base + j]], msg_v.at[j],
                             sems.at[j])
            for j in range(KD)
        ]
        for j in range(KD):
            descs[j].wait()
            pltpu.sync_copy(msg_v.at[j], acc.at[dst_v.at[base + j]], add=True)
        return carry

    lax.fori_loop(0, GPW_NARROW // KD, chunk, 0)
    plsc.subcore_barrier()
    _flush_acc(acc, out_hbm, c, s)


def _make_narrow(W):
    @functools.partial(
        pl.kernel,
        out_type=jax.ShapeDtypeStruct((2, NACC, W), jnp.float32),
        mesh=_MESH,
        scratch_types=[
            pltpu.VMEM((GPW_NARROW, 128), jnp.int32),
            pltpu.VMEM((GPW_NARROW, 128), jnp.int32),
            pltpu.VMEM((KD, 128, W), jnp.float32),
            pltpu.VMEM_SHARED((NACC, W), jnp.float32),
            pltpu.SemaphoreType.DMA((KD,)),
        ],
    )
    def kern(src_hbm, dst_hbm, table_hbm, zeros_hbm, out_hbm,
             src_v, dst_v, msg_v, acc, sems):
        _narrow_body(src_hbm, dst_hbm, table_hbm, zeros_hbm, out_hbm,
                     src_v, dst_v, msg_v, acc, sems)

    return kern


_sc_narrow2 = _make_narrow(2)
_sc_narrow4 = _make_narrow(4)

_HALF_GPW = GPW_WIDE // 2


@functools.partial(
    pl.kernel,
    out_type=jax.ShapeDtypeStruct((2, NACC, 32), jnp.float32),
    mesh=_MESH,
    scratch_types=[
        pltpu.VMEM((_HALF_GPW, 128), jnp.int32),
        pltpu.VMEM((_HALF_GPW, 128), jnp.int32),
        pltpu.VMEM((KDW, 128, 32), jnp.float32),
        pltpu.VMEM_SHARED((NACC, 32), jnp.float32),
        pltpu.SemaphoreType.DMA((KDW,)),
    ],
)
def _sc_wide(src_a_hbm, src_b_hbm, dst_hbm, table_hbm, zeros_hbm, out_hbm,
             src_v, dst_v, msg_v, acc, sems):
    c = lax.axis_index("c")
    s = lax.axis_index("s")
    _zero_acc(zeros_hbm, acc, s)
    plsc.subcore_barrier()

    for half in range(2):
        g0 = s * GPW_WIDE + half * _HALF_GPW

        @pl.when(c == 0)
        def _():
            pltpu.sync_copy(src_a_hbm.at[pl.ds(g0, _HALF_GPW)], src_v)

        @pl.when(c == 1)
        def _():
            pltpu.sync_copy(src_b_hbm.at[pl.ds(g0, _HALF_GPW)], src_v)

        pltpu.sync_copy(dst_hbm.at[pl.ds(g0, _HALF_GPW)], dst_v)

        def chunk(cix, carry):
            base = cix * KDW
            descs = [
                pltpu.async_copy(table_hbm.at[src_v.at[base + j]],
                                 msg_v.at[j], sems.at[j])
                for j in range(KDW)
            ]
            for j in range(KDW):
                descs[j].wait()
                pltpu.sync_copy(msg_v.at[j], acc.at[dst_v.at[base + j]],
                                add=True)
            return carry

        lax.fori_loop(0, _HALF_GPW // KDW, chunk, 0)

    plsc.subcore_barrier()
    _flush_acc(acc, out_hbm, c, s)


def _row_spec(width):
    return pl.BlockSpec((BLK, width), lambda i: (i, 0))


def _pair_spec(width):
    return pl.BlockSpec((2, BLK, width), lambda i: (0, i, 0))


def _full_spec(r, cols):
    return pl.BlockSpec((r, cols), lambda i: (0, 0))


def _tc_shift_body(src_ref, out_ref):
    out_ref[...] = src_ref[...] + NACC


def _tc_shift(src_g):
    return pl.pallas_call(
        _tc_shift_body,
        grid=(16,),
        in_specs=[pl.BlockSpec((NG // 16, 128), lambda i: (i, 0))],
        out_specs=pl.BlockSpec((NG // 16, 128), lambda i: (i, 0)),
        out_shape=jax.ShapeDtypeStruct((NG, 128), jnp.int32),
    )(src_g)


def _tc_a_body(cnt_ref, x_ref, dis_ref, dinv_ref, g0_ref):
    deg = cnt_ref[0] + cnt_ref[1] + 1.0
    dis = lax.rsqrt(deg)
    dis_ref[...] = dis
    dinv_ref[...] = 1.0 / deg
    g0_ref[...] = jnp.concatenate([dis * x_ref[...], dis], axis=1)


def _tc_a(cntp, xp):
    return pl.pallas_call(
        _tc_a_body,
        grid=(GRID,),
        in_specs=[_pair_spec(1), _row_spec(1)],
        out_specs=[_row_spec(1), _row_spec(1), _row_spec(2)],
        out_shape=[
            jax.ShapeDtypeStruct((NACC, 1), jnp.float32),
            jax.ShapeDtypeStruct((NACC, 1), jnp.float32),
            jax.ShapeDtypeStruct((NACC, 2), jnp.float32),
        ],
    )(cntp, xp)


def _tc_b_body(s0_ref, x_ref, dis_ref, dinv_ref, delta_ref, w0_ref, b0_ref,
               h1_ref, g1_ref):
    dis = dis_ref[...]
    dinv = dinv_ref[...]
    s0 = s0_ref[0] + s0_ref[1]
    a0 = dis * s0[:, 0:1] + dinv * x_ref[...]
    a1 = dis * s0[:, 1:2] + dinv
    w0d = jnp.sum(delta_ref[...].reshape(4, 1) * w0_ref[1:5, :], axis=0,
                  keepdims=True)
    z0 = a0 * w0_ref[0:1, :] + a1 * w0d + b0_ref[...]
    h1 = jnp.maximum(z0, 0.0)
    h1_ref[...] = h1
    g1 = dis * h1
    g1_ref[0] = g1[:, 0:32]
    g1_ref[1] = g1[:, 32:64]


def _tc_b(s0p, xp, dis, dinv, delta, W0, b0):
    return pl.pallas_call(
        _tc_b_body,
        grid=(GRID,),
        in_specs=[_pair_spec(2), _row_spec(1), _row_spec(1),
                  _full_spec(1, 4), _full_spec(5, 64), _full_spec(1, 64)],
        out_specs=[_row_spec(64), _pair_spec(32)],
        out_shape=[
            jax.ShapeDtypeStruct((NACC, 64), jnp.float32),
            jax.ShapeDtypeStruct((2, NACC, 32), jnp.float32),
        ],
    )(s0p, xp, dis, dinv, delta.reshape(1, 4), W0, b0.reshape(1, 64))


def _tc_c_body(s1_ref, h1_ref, dis_ref, dinv_ref, w1_ref, b1_ref, w2_ref,
               m2_ref, g2_ref):
    dis = dis_ref[...]
    dinv = dinv_ref[...]
    agg = jnp.concatenate([s1_ref[0], s1_ref[1]], axis=1)
    t = dis * agg + dinv * h1_ref[...]
    z1 = jnp.dot(t, w1_ref[...], preferred_element_type=jnp.float32)
    h2 = jnp.maximum(z1 + b1_ref[...], 0.0)
    m2 = jnp.dot(h2, w2_ref[...], preferred_element_type=jnp.float32)
    m2_ref[...] = m2
    g2_ref[...] = dis * m2


def _tc_c(s1, h1, dis, dinv, W1, b1, W2p):
    return pl.pallas_call(
        _tc_c_body,
        grid=(GRID,),
        in_specs=[_pair_spec(32), _row_spec(64), _row_spec(1), _row_spec(1),
                  _full_spec(64, 64), _full_spec(1, 64), _full_spec(64, 4)],
        out_specs=[_row_spec(4), _row_spec(4)],
        out_shape=[
            jax.ShapeDtypeStruct((NACC, 4), jnp.float32),
            jax.ShapeDtypeStruct((NACC, 4), jnp.float32),
        ],
    )(s1, h1, dis, dinv, W1, b1.reshape(1, 64), W2p)


def _tc_d_body(s2_ref, m2_ref, dis_ref, dinv_ref, b2_ref, x_ref, pos_ref,
               ntype_ref, npos_ref):
    o = (dis_ref[...] * (s2_ref[0] + s2_ref[1])
         + dinv_ref[...] * m2_ref[...] + b2_ref[...])
    npos_ref[...] = pos_ref[...] + o[:, 0:2]
    ntype_ref[...] = x_ref[...] + o[:, 2:3]


def _tc_d(s2p, m2, dis, dinv, b2p, xp, posp):
    return pl.pallas_call(
        _tc_d_body,
        grid=(GRID,),
        in_specs=[_pair_spec(4), _row_spec(4), _row_spec(1), _row_spec(1),
                  _full_spec(1, 4), _row_spec(1), _row_spec(2)],
        out_specs=[_row_spec(1), _row_spec(2)],
        out_shape=[
            jax.ShapeDtypeStruct((NACC, 1), jnp.float32),
            jax.ShapeDtypeStruct((NACC, 2), jnp.float32),
        ],
    )(s2p, m2, dis, dinv, b2p, xp, posp)


def kernel(x, pos, delta, edge_index, W0, b0, W1, b1, W2, b2):
    src = edge_index[0].astype(jnp.int32)
    dst = edge_index[1].astype(jnp.int32)
    pad = EPAD - E
    src_g = jnp.concatenate([src, jnp.zeros((pad,), jnp.int32)]).reshape(NG, 128)
    dst_g = jnp.concatenate([dst, jnp.full((pad,), TRASH, jnp.int32)]).reshape(NG, 128)
    xp = jnp.pad(x, ((0, NACC - N), (0, 0)))
    posp = jnp.pad(pos, ((0, NACC - N), (0, 0)))
    zeros1 = jnp.zeros((NACC, 1), jnp.float32)
    zeros2 = jnp.zeros((NACC, 2), jnp.float32)
    zeros4 = jnp.zeros((NACC, 4), jnp.float32)
    zeros32 = jnp.zeros((NACC, 32), jnp.float32)
    ones1 = jnp.ones((128, 1), jnp.float32)
    W2p = jnp.pad(W2, ((0, 0), (0, 1)))
    b2p = jnp.pad(b2, (0, 1)).reshape(1, 4)

    src_shift = _tc_shift(src_g)
    cntp = _sc_degree(dst_g, ones1, zeros1)
    dis, dinv, g0 = _tc_a(cntp, xp)
    s0p = _sc_narrow2(src_g, dst_g, g0, zeros2)
    h1, g1 = _tc_b(s0p, xp, dis, dinv, delta, W0, b0)
    s1 = _sc_wide(src_g, src_shift, dst_g, g1.reshape(2 * NACC, 32), zeros32)
    m2, g2 = _tc_c(s1, h1, dis, dinv, W1, b1, W2p)
    s2p = _sc_narrow4(src_g, dst_g, g2, zeros4)
    ntype, npos = _tc_d(s2p, m2, dis, dinv, b2p, xp, posp)
    return (ntype[:N], npos[:N])


# speedup vs baseline: 19.4581x; 19.4581x over previous
import functools

import jax
import jax.numpy as jnp
from jax import lax
from jax.experimental import pallas as pl
from jax.experimental.pallas import tpu as pltpu
from jax.experimental.pallas import tpu_sc as plsc

N = 50000
NACC = 50048
ROWS_PT = NACC // 16
E = 800000
EPAD = 819200
NG = EPAD // 128
TRASH = N
GPW_NARROW = NG // 32
GPW_WIDE = NG // 16
KD = 8
KDW = 4
WN = 16
BLK = 2176
GRID = NACC // BLK

_MESH = plsc.VectorSubcoreMesh(core_axis_name="c", subcore_axis_name="s")
_CP = pltpu.CompilerParams(use_tc_tiling_on_sc=False)


def _zero_acc(zeros_hbm, acc, s):
    r0 = s * ROWS_PT
    pltpu.sync_copy(zeros_hbm.at[pl.ds(r0, ROWS_PT)], acc.at[pl.ds(r0, ROWS_PT)])


def _flush_acc(acc, out_hbm, c, s):
    r0 = s * ROWS_PT
    pltpu.sync_copy(acc.at[pl.ds(r0, ROWS_PT)], out_hbm.at[c, pl.ds(r0, ROWS_PT)])


@functools.partial(
    pl.kernel,
    out_type=jax.ShapeDtypeStruct((2, NACC, WN), jnp.float32),
    mesh=_MESH,
    compiler_params=_CP,
    scratch_types=[
        pltpu.VMEM_SHARED((NACC, WN), jnp.float32),
        pltpu.VMEM((GPW_NARROW, 128), jnp.int32),
        pltpu.VMEM((128, WN), jnp.float32),
    ],
)
def _sc_degree(dst_hbm, ones_hbm, zeros_hbm, out_hbm, acc, dst_v, ones_v):
    c = lax.axis_index("c")
    s = lax.axis_index("s")
    wid = c * 16 + s
    _zero_acc(zeros_hbm, acc, s)
    pltpu.sync_copy(ones_hbm, ones_v)
    pltpu.sync_copy(dst_hbm.at[pl.ds(wid * GPW_NARROW, GPW_NARROW)], dst_v)
    plsc.subcore_barrier()

    def body(g, carry):
        pltpu.sync_copy(ones_v, acc.at[dst_v.at[g]], add=True)
        return carry

    lax.fori_loop(0, GPW_NARROW, body, 0)
    plsc.subcore_barrier()
    _flush_acc(acc, out_hbm, c, s)


def _make_narrow():
    @functools.partial(
        pl.kernel,
        out_type=jax.ShapeDtypeStruct((2, NACC, WN), jnp.float32),
        mesh=_MESH,
        compiler_params=_CP,
        scratch_types=[
            pltpu.VMEM_SHARED((NACC, WN), jnp.float32),
            pltpu.VMEM((GPW_NARROW, 128), jnp.int32),
            pltpu.VMEM((GPW_NARROW, 128), jnp.int32),
            pltpu.VMEM((KD, 128, WN), jnp.float32),
            pltpu.SemaphoreType.DMA((KD,)),
        ],
    )
    def kern(src_hbm, dst_hbm, table_hbm, zeros_hbm, out_hbm,
             acc, src_v, dst_v, msg_v, sems):
        c = lax.axis_index("c")
        s = lax.axis_index("s")
        wid = c * 16 + s
        g0 = wid * GPW_NARROW
        _zero_acc(zeros_hbm, acc, s)
        pltpu.sync_copy(src_hbm.at[pl.ds(g0, GPW_NARROW)], src_v)
        pltpu.sync_copy(dst_hbm.at[pl.ds(g0, GPW_NARROW)], dst_v)
        plsc.subcore_barrier()

        def chunk(cix, carry):
            base = cix * KD
            descs = [
                pltpu.async_copy(table_hbm.at[src_v.at[base + j]], msg_v.at[j],
                                 sems.at[j])
                for j in range(KD)
            ]
            for j in range(KD):
                descs[j].wait()
                pltpu.sync_copy(msg_v.at[j], acc.at[dst_v.at[base + j]],
                                add=True)
            return carry

        lax.fori_loop(0, GPW_NARROW // KD, chunk, 0)
        plsc.subcore_barrier()
        _flush_acc(acc, out_hbm, c, s)

    return kern


_sc_narrow_a = _make_narrow()
_sc_narrow_b = _make_narrow()

ROWS_REF = 20
NREF = GPW_WIDE // ROWS_REF


@functools.partial(
    pl.kernel,
    out_type=jax.ShapeDtypeStruct((2, NACC, 32), jnp.float32),
    mesh=_MESH,
    compiler_params=_CP,
    scratch_types=[
        pltpu.VMEM_SHARED((NACC, 32), jnp.float32),
        pltpu.VMEM((ROWS_REF, 128), jnp.int32),
        pltpu.VMEM((ROWS_REF, 128), jnp.int32),
        pltpu.VMEM((KDW, 128, 32), jnp.float32),
        pltpu.SemaphoreType.DMA((KDW,)),
    ],
)
def _sc_wide(src_a_hbm, src_b_hbm, dst_hbm, table_hbm, zeros_hbm, out_hbm,
             acc, src_v, dst_v, msg_v, sems):
    c = lax.axis_index("c")
    s = lax.axis_index("s")
    _zero_acc(zeros_hbm, acc, s)
    plsc.subcore_barrier()

    def refill(r, carry):
        g0 = s * GPW_WIDE + r * ROWS_REF

        @pl.when(c == 0)
        def _():
            pltpu.sync_copy(src_a_hbm.at[pl.ds(g0, ROWS_REF)], src_v)

        @pl.when(c == 1)
        def _():
            pltpu.sync_copy(src_b_hbm.at[pl.ds(g0, ROWS_REF)], src_v)

        pltpu.sync_copy(dst_hbm.at[pl.ds(g0, ROWS_REF)], dst_v)

        def chunk(cix, carry2):
            base = cix * KDW
            descs = [
                pltpu.async_copy(table_hbm.at[src_v.at[base + j]],
                                 msg_v.at[j], sems.at[j])
                for j in range(KDW)
            ]
            for j in range(KDW):
                descs[j].wait()
                pltpu.sync_copy(msg_v.at[j], acc.at[dst_v.at[base + j]],
                                add=True)
            return carry2

        lax.fori_loop(0, ROWS_REF // KDW, chunk, 0)
        return carry

    lax.fori_loop(0, NREF, refill, 0)
    plsc.subcore_barrier()
    _flush_acc(acc, out_hbm, c, s)


def _row_spec(width):
    return pl.BlockSpec((BLK, width), lambda i: (i, 0))


def _pair_spec(width):
    return pl.BlockSpec((2, BLK, width), lambda i: (0, i, 0))


def _full_spec(r, cols):
    return pl.BlockSpec((r, cols), lambda i: (0, 0))


def _tc_shift_body(src_ref, out_ref):
    out_ref[...] = src_ref[...] + NACC


def _tc_shift(src_g):
    return pl.pallas_call(
        _tc_shift_body,
        grid=(16,),
        in_specs=[pl.BlockSpec((NG // 16, 128), lambda i: (i, 0))],
        out_specs=pl.BlockSpec((NG // 16, 128), lambda i: (i, 0)),
        out_shape=jax.ShapeDtypeStruct((NG, 128), jnp.int32),
    )(src_g)


def _tc_a_body(cnt_ref, x_ref, dis_ref, dinv_ref, g0_ref):
    deg = cnt_ref[0, :, 0:1] + cnt_ref[1, :, 0:1] + 1.0
    dis = lax.rsqrt(deg)
    dis_ref[...] = dis
    dinv_ref[...] = 1.0 / deg
    pad = jnp.zeros((BLK, WN - 2), jnp.float32)
    g0_ref[...] = jnp.concatenate([dis * x_ref[...], dis, pad], axis=1)


def _tc_a(cntp, xp):
    return pl.pallas_call(
        _tc_a_body,
        grid=(GRID,),
        in_specs=[_pair_spec(WN), _row_spec(1)],
        out_specs=[_row_spec(1), _row_spec(1), _row_spec(WN)],
        out_shape=[
            jax.ShapeDtypeStruct((NACC, 1), jnp.float32),
            jax.ShapeDtypeStruct((NACC, 1), jnp.float32),
            jax.ShapeDtypeStruct((NACC, WN), jnp.float32),
        ],
    )(cntp, xp)


def _tc_b_body(s0_ref, x_ref, dis_ref, dinv_ref, delta_ref, w0_ref, b0_ref,
               h1_ref, g1_ref):
    dis = dis_ref[...]
    dinv = dinv_ref[...]
    s0 = s0_ref[0] + s0_ref[1]
    a0 = dis * s0[:, 0:1] + dinv * x_ref[...]
    a1 = dis * s0[:, 1:2] + dinv
    w0d = jnp.sum(delta_ref[...].reshape(4, 1) * w0_ref[1:5, :], axis=0,
                  keepdims=True)
    z0 = a0 * w0_ref[0:1, :] + a1 * w0d + b0_ref[...]
    h1 = jnp.maximum(z0, 0.0)
    h1_ref[...] = h1
    g1 = dis * h1
    g1_ref[0] = g1[:, 0:32]
    g1_ref[1] = g1[:, 32:64]


def _tc_b(s0p, xp, dis, dinv, delta, W0, b0):
    return pl.pallas_call(
        _tc_b_body,
        grid=(GRID,),
        in_specs=[_pair_spec(WN), _row_spec(1), _row_spec(1), _row_spec(1),
                  _full_spec(1, 4), _full_spec(5, 64), _full_spec(1, 64)],
        out_specs=[_row_spec(64), _pair_spec(32)],
        out_shape=[
            jax.ShapeDtypeStruct((NACC, 64), jnp.float32),
            jax.ShapeDtypeStruct((2, NACC, 32), jnp.float32),
        ],
    )(s0p, xp, dis, dinv, delta.reshape(1, 4), W0, b0.reshape(1, 64))


def _tc_c_body(s1_ref, h1_ref, dis_ref, dinv_ref, w1_ref, b1_ref, w2_ref,
               m2_ref, g2_ref):
    dis = dis_ref[...]
    dinv = dinv_ref[...]
    agg = jnp.concatenate([s1_ref[0], s1_ref[1]], axis=1)
    t = dis * agg + dinv * h1_ref[...]
    z1 = jnp.dot(t, w1_ref[...], preferred_element_type=jnp.float32)
    h2 = jnp.maximum(z1 + b1_ref[...], 0.0)
    m2 = jnp.dot(h2, w2_ref[...], preferred_element_type=jnp.float32)
    m2_ref[...] = m2
    pad = jnp.zeros((BLK, WN - 4), jnp.float32)
    g2_ref[...] = jnp.concatenate([dis * m2, pad], axis=1)


def _tc_c(s1, h1, dis, dinv, W1, b1, W2p):
    return pl.pallas_call(
        _tc_c_body,
        grid=(GRID,),
        in_specs=[_pair_spec(32), _row_spec(64), _row_spec(1), _row_spec(1),
                  _full_spec(64, 64), _full_spec(1, 64), _full_spec(64, 4)],
        out_specs=[_row_spec(4), _row_spec(WN)],
        out_shape=[
            jax.ShapeDtypeStruct((NACC, 4), jnp.float32),
            jax.ShapeDtypeStruct((NACC, WN), jnp.float32),
        ],
    )(s1, h1, dis, dinv, W1, b1.reshape(1, 64), W2p)


def _tc_d_body(s2_ref, m2_ref, dis_ref, dinv_ref, b2_ref, x_ref, pos_ref,
               ntype_ref, npos_ref):
    o = (dis_ref[...] * (s2_ref[0, :, 0:4] + s2_ref[1, :, 0:4])
         + dinv_ref[...] * m2_ref[...] + b2_ref[...])
    npos_ref[...] = pos_ref[...] + o[:, 0:2]
    ntype_ref[...] = x_ref[...] + o[:, 2:3]


def _tc_d(s2p, m2, dis, dinv, b2p, xp, posp):
    return pl.pallas_call(
        _tc_d_body,
        grid=(GRID,),
        in_specs=[_pair_spec(WN), _row_spec(4), _row_spec(1), _row_spec(1),
                  _full_spec(1, 4), _row_spec(1), _row_spec(2)],
        out_specs=[_row_spec(1), _row_spec(2)],
        out_shape=[
            jax.ShapeDtypeStruct((NACC, 1), jnp.float32),
            jax.ShapeDtypeStruct((NACC, 2), jnp.float32),
        ],
    )(s2p, m2, dis, dinv, b2p, xp, posp)


def kernel(x, pos, delta, edge_index, W0, b0, W1, b1, W2, b2):
    src = edge_index[0].astype(jnp.int32)
    dst = edge_index[1].astype(jnp.int32)
    pad = EPAD - E
    src_g = jnp.concatenate([src, jnp.zeros((pad,), jnp.int32)]).reshape(NG, 128)
    dst_g = jnp.concatenate([dst, jnp.full((pad,), TRASH, jnp.int32)]).reshape(NG, 128)
    xp = jnp.pad(x, ((0, NACC - N), (0, 0)))
    posp = jnp.pad(pos, ((0, NACC - N), (0, 0)))
    zeros16 = jnp.zeros((NACC, WN), jnp.float32)
    zeros32 = jnp.zeros((NACC, 32), jnp.float32)
    ones16 = jnp.ones((128, WN), jnp.float32)
    W2p = jnp.pad(W2, ((0, 0), (0, 1)))
    b2p = jnp.pad(b2, (0, 1)).reshape(1, 4)

    src_shift = _tc_shift(src_g)
    cntp = _sc_degree(dst_g, ones16, zeros16)
    dis, dinv, g0 = _tc_a(cntp, xp)
    s0p = _sc_narrow_a(src_g, dst_g, g0, zeros16)
    h1, g1 = _tc_b(s0p, xp, dis, dinv, delta, W0, b0)
    s1 = _sc_wide(src_g, src_shift, dst_g, g1.reshape(2 * NACC, 32), zeros32)
    m2, g2 = _tc_c(s1, h1, dis, dinv, W1, b1, W2p)
    s2p = _sc_narrow_b(src_g, dst_g, g2, zeros16)
    ntype, npos = _tc_d(s2p, m2, dis, dinv, b2p, xp, posp)
    return (ntype[:N], npos[:N])


# speedup vs baseline: 20.7861x; 1.0683x over previous
import functools

import jax
import jax.numpy as jnp
from jax import lax
from jax.experimental import pallas as pl
from jax.experimental.pallas import tpu as pltpu
from jax.experimental.pallas import tpu_sc as plsc

N = 50000
NACC = 50048
ROWS_PT = NACC // 16
E = 800000
EPAD = 819200
NG = EPAD // 128
TRASH = N
GPW_NARROW = NG // 32
GPW_WIDE = NG // 16
KD = 10
KDW = 4
WN = 16
BLK = 2176
GRID = NACC // BLK

_MESH = plsc.VectorSubcoreMesh(core_axis_name="c", subcore_axis_name="s")
_CP = pltpu.CompilerParams(use_tc_tiling_on_sc=False)


def _zero_acc(zeros_hbm, acc, s):
    r0 = s * ROWS_PT
    pltpu.sync_copy(zeros_hbm.at[pl.ds(r0, ROWS_PT)], acc.at[pl.ds(r0, ROWS_PT)])


def _flush_acc(acc, out_hbm, c, s):
    r0 = s * ROWS_PT
    pltpu.sync_copy(acc.at[pl.ds(r0, ROWS_PT)], out_hbm.at[c, pl.ds(r0, ROWS_PT)])


@functools.partial(
    pl.kernel,
    out_type=jax.ShapeDtypeStruct((2, NACC, WN), jnp.float32),
    mesh=_MESH,
    compiler_params=_CP,
    scratch_types=[
        pltpu.VMEM_SHARED((NACC, WN), jnp.float32),
        pltpu.VMEM((GPW_NARROW, 128), jnp.int32),
        pltpu.VMEM((128, WN), jnp.float32),
        pltpu.SemaphoreType.DMA,
    ],
)
def _sc_degree(dst_hbm, ones_hbm, zeros_hbm, out_hbm, acc, dst_v, ones_v, sem):
    c = lax.axis_index("c")
    s = lax.axis_index("s")
    wid = c * 16 + s
    _zero_acc(zeros_hbm, acc, s)
    pltpu.sync_copy(ones_hbm, ones_v)
    pltpu.sync_copy(dst_hbm.at[pl.ds(wid * GPW_NARROW, GPW_NARROW)], dst_v)
    plsc.subcore_barrier()

    def body(g, carry):
        pltpu.async_copy(ones_v, acc.at[dst_v.at[g]], sem, add=True)
        return carry

    lax.fori_loop(0, GPW_NARROW, body, 0)

    def drain(g, carry):
        pltpu.make_async_copy(zeros_hbm.at[pl.ds(0, 128)], ones_v, sem).wait()
        return carry

    lax.fori_loop(0, GPW_NARROW, drain, 0)
    plsc.subcore_barrier()
    _flush_acc(acc, out_hbm, c, s)


def _make_narrow():
    @functools.partial(
        pl.kernel,
        out_type=jax.ShapeDtypeStruct((2, NACC, WN), jnp.float32),
        mesh=_MESH,
        compiler_params=_CP,
        scratch_types=[
            pltpu.VMEM_SHARED((NACC, WN), jnp.float32),
            pltpu.VMEM((GPW_NARROW, 128), jnp.int32),
            pltpu.VMEM((GPW_NARROW, 128), jnp.int32),
            pltpu.VMEM((KD, 128, WN), jnp.float32),
            pltpu.SemaphoreType.DMA((KD,)),
        ],
    )
    def kern(src_hbm, dst_hbm, table_hbm, zeros_hbm, out_hbm,
             acc, src_v, dst_v, msg_v, sems):
        c = lax.axis_index("c")
        s = lax.axis_index("s")
        wid = c * 16 + s
        g0 = wid * GPW_NARROW
        _zero_acc(zeros_hbm, acc, s)
        pltpu.sync_copy(src_hbm.at[pl.ds(g0, GPW_NARROW)], src_v)
        pltpu.sync_copy(dst_hbm.at[pl.ds(g0, GPW_NARROW)], dst_v)
        plsc.subcore_barrier()

        for j in range(KD):
            pltpu.async_copy(table_hbm.at[src_v.at[j]], msg_v.at[j],
                             sems.at[j])

        def chunk(cix, carry):
            base = cix * KD
            for j in range(KD):
                pltpu.make_async_copy(zeros_hbm.at[pl.ds(0, 128)],
                                      msg_v.at[j], sems.at[j]).wait()
                pltpu.sync_copy(msg_v.at[j], acc.at[dst_v.at[base + j]],
                                add=True)

                @pl.when(base + KD + j < GPW_NARROW)
                def _():
                    pltpu.async_copy(table_hbm.at[src_v.at[base + KD + j]],
                                     msg_v.at[j], sems.at[j])
            return carry

        lax.fori_loop(0, GPW_NARROW // KD, chunk, 0)
        plsc.subcore_barrier()
        _flush_acc(acc, out_hbm, c, s)

    return kern


_sc_narrow_a = _make_narrow()
_sc_narrow_b = _make_narrow()

ROWS_REF = 20
NREF = GPW_WIDE // ROWS_REF


@functools.partial(
    pl.kernel,
    out_type=jax.ShapeDtypeStruct((2, NACC, 32), jnp.float32),
    mesh=_MESH,
    compiler_params=_CP,
    scratch_types=[
        pltpu.VMEM_SHARED((NACC, 32), jnp.float32),
        pltpu.VMEM((ROWS_REF, 128), jnp.int32),
        pltpu.VMEM((ROWS_REF, 128), jnp.int32),
        pltpu.VMEM((KDW, 128, 32), jnp.float32),
        pltpu.SemaphoreType.DMA((KDW,)),
    ],
)
def _sc_wide(src_a_hbm, src_b_hbm, dst_hbm, table_hbm, zeros_hbm, out_hbm,
             acc, src_v, dst_v, msg_v, sems):
    c = lax.axis_index("c")
    s = lax.axis_index("s")
    _zero_acc(zeros_hbm, acc, s)
    plsc.subcore_barrier()

    def refill(r, carry):
        g0 = s * GPW_WIDE + r * ROWS_REF

        @pl.when(c == 0)
        def _():
            pltpu.sync_copy(src_a_hbm.at[pl.ds(g0, ROWS_REF)], src_v)

        @pl.when(c == 1)
        def _():
            pltpu.sync_copy(src_b_hbm.at[pl.ds(g0, ROWS_REF)], src_v)

        pltpu.sync_copy(dst_hbm.at[pl.ds(g0, ROWS_REF)], dst_v)

        for j in range(KDW):
            pltpu.async_copy(table_hbm.at[src_v.at[j]], msg_v.at[j],
                             sems.at[j])

        def chunk(cix, carry2):
            base = cix * KDW
            for j in range(KDW):
                pltpu.make_async_copy(zeros_hbm.at[pl.ds(0, 128)],
                                      msg_v.at[j], sems.at[j]).wait()
                pltpu.sync_copy(msg_v.at[j], acc.at[dst_v.at[base + j]],
                                add=True)

                @pl.when(base + KDW + j < ROWS_REF)
                def _():
                    pltpu.async_copy(table_hbm.at[src_v.at[base + KDW + j]],
                                     msg_v.at[j], sems.at[j])
            return carry2

        lax.fori_loop(0, ROWS_REF // KDW, chunk, 0)
        return carry

    lax.fori_loop(0, NREF, refill, 0)
    plsc.subcore_barrier()
    _flush_acc(acc, out_hbm, c, s)


def _row_spec(width):
    return pl.BlockSpec((BLK, width), lambda i: (i, 0))


def _pair_spec(width):
    return pl.BlockSpec((2, BLK, width), lambda i: (0, i, 0))


def _full_spec(r, cols):
    return pl.BlockSpec((r, cols), lambda i: (0, 0))


def _tc_shift_body(src_ref, out_ref):
    out_ref[...] = src_ref[...] + NACC


def _tc_shift(src_g):
    return pl.pallas_call(
        _tc_shift_body,
        grid=(16,),
        in_specs=[pl.BlockSpec((NG // 16, 128), lambda i: (i, 0))],
        out_specs=pl.BlockSpec((NG // 16, 128), lambda i: (i, 0)),
        out_shape=jax.ShapeDtypeStruct((NG, 128), jnp.int32),
    )(src_g)


def _tc_a_body(cnt_ref, x_ref, dis_ref, dinv_ref, g0_ref):
    deg = cnt_ref[0, :, 0:1] + cnt_ref[1, :, 0:1] + 1.0
    dis = lax.rsqrt(deg)
    dis_ref[...] = dis
    dinv_ref[...] = 1.0 / deg
    pad = jnp.zeros((BLK, WN - 2), jnp.float32)
    g0_ref[...] = jnp.concatenate([dis * x_ref[...], dis, pad], axis=1)


def _tc_a(cntp, xp):
    return pl.pallas_call(
        _tc_a_body,
        grid=(GRID,),
        in_specs=[_pair_spec(WN), _row_spec(1)],
        out_specs=[_row_spec(1), _row_spec(1), _row_spec(WN)],
        out_shape=[
            jax.ShapeDtypeStruct((NACC, 1), jnp.float32),
            jax.ShapeDtypeStruct((NACC, 1), jnp.float32),
            jax.ShapeDtypeStruct((NACC, WN), jnp.float32),
        ],
    )(cntp, xp)


def _tc_b_body(s0_ref, x_ref, dis_ref, dinv_ref, delta_ref, w0_ref, b0_ref,
               h1_ref, g1_ref):
    dis = dis_ref[...]
    dinv = dinv_ref[...]
    s0 = s0_ref[0] + s0_ref[1]
    a0 = dis * s0[:, 0:1] + dinv * x_ref[...]
    a1 = dis * s0[:, 1:2] + dinv
    w0d = jnp.sum(delta_ref[...].reshape(4, 1) * w0_ref[1:5, :], axis=0,
                  keepdims=True)
    z0 = a0 * w0_ref[0:1, :] + a1 * w0d + b0_ref[...]
    h1 = jnp.maximum(z0, 0.0)
    h1_ref[...] = h1
    g1 = dis * h1
    g1_ref[0] = g1[:, 0:32]
    g1_ref[1] = g1[:, 32:64]


def _tc_b(s0p, xp, dis, dinv, delta, W0, b0):
    return pl.pallas_call(
        _tc_b_body,
        grid=(GRID,),
        in_specs=[_pair_spec(WN), _row_spec(1), _row_spec(1), _row_spec(1),
                  _full_spec(1, 4), _full_spec(5, 64), _full_spec(1, 64)],
        out_specs=[_row_spec(64), _pair_spec(32)],
        out_shape=[
            jax.ShapeDtypeStruct((NACC, 64), jnp.float32),
            jax.ShapeDtypeStruct((2, NACC, 32), jnp.float32),
        ],
    )(s0p, xp, dis, dinv, delta.reshape(1, 4), W0, b0.reshape(1, 64))


def _tc_c_body(s1_ref, h1_ref, dis_ref, dinv_ref, w1_ref, b1_ref, w2_ref,
               m2_ref, g2_ref):
    dis = dis_ref[...]
    dinv = dinv_ref[...]
    agg = jnp.concatenate([s1_ref[0], s1_ref[1]], axis=1)
    t = dis * agg + dinv * h1_ref[...]
    z1 = jnp.dot(t, w1_ref[...], preferred_element_type=jnp.float32)
    h2 = jnp.maximum(z1 + b1_ref[...], 0.0)
    m2 = jnp.dot(h2, w2_ref[...], preferred_element_type=jnp.float32)
    m2_ref[...] = m2
    pad = jnp.zeros((BLK, WN - 4), jnp.float32)
    g2_ref[...] = jnp.concatenate([dis * m2, pad], axis=1)


def _tc_c(s1, h1, dis, dinv, W1, b1, W2p):
    return pl.pallas_call(
        _tc_c_body,
        grid=(GRID,),
        in_specs=[_pair_spec(32), _row_spec(64), _row_spec(1), _row_spec(1),
                  _full_spec(64, 64), _full_spec(1, 64), _full_spec(64, 4)],
        out_specs=[_row_spec(4), _row_spec(WN)],
        out_shape=[
            jax.ShapeDtypeStruct((NACC, 4), jnp.float32),
            jax.ShapeDtypeStruct((NACC, WN), jnp.float32),
        ],
    )(s1, h1, dis, dinv, W1, b1.reshape(1, 64), W2p)


def _tc_d_body(s2_ref, m2_ref, dis_ref, dinv_ref, b2_ref, x_ref, pos_ref,
               ntype_ref, npos_ref):
    o = (dis_ref[...] * (s2_ref[0, :, 0:4] + s2_ref[1, :, 0:4])
         + dinv_ref[...] * m2_ref[...] + b2_ref[...])
    npos_ref[...] = pos_ref[...] + o[:, 0:2]
    ntype_ref[...] = x_ref[...] + o[:, 2:3]


def _tc_d(s2p, m2, dis, dinv, b2p, xp, posp):
    return pl.pallas_call(
        _tc_d_body,
        grid=(GRID,),
        in_specs=[_pair_spec(WN), _row_spec(4), _row_spec(1), _row_spec(1),
                  _full_spec(1, 4), _row_spec(1), _row_spec(2)],
        out_specs=[_row_spec(1), _row_spec(2)],
        out_shape=[
            jax.ShapeDtypeStruct((NACC, 1), jnp.float32),
            jax.ShapeDtypeStruct((NACC, 2), jnp.float32),
        ],
    )(s2p, m2, dis, dinv, b2p, xp, posp)


def kernel(x, pos, delta, edge_index, W0, b0, W1, b1, W2, b2):
    src = edge_index[0].astype(jnp.int32)
    dst = edge_index[1].astype(jnp.int32)
    pad = EPAD - E
    src_g = jnp.concatenate([src, jnp.zeros((pad,), jnp.int32)]).reshape(NG, 128)
    dst_g = jnp.concatenate([dst, jnp.full((pad,), TRASH, jnp.int32)]).reshape(NG, 128)
    xp = jnp.pad(x, ((0, NACC - N), (0, 0)))
    posp = jnp.pad(pos, ((0, NACC - N), (0, 0)))
    zeros16 = jnp.zeros((NACC, WN), jnp.float32)
    zeros32 = jnp.zeros((NACC, 32), jnp.float32)
    ones16 = jnp.ones((128, WN), jnp.float32)
    W2p = jnp.pad(W2, ((0, 0), (0, 1)))
    b2p = jnp.pad(b2, (0, 1)).reshape(1, 4)

    src_shift = _tc_shift(src_g)
    cntp = _sc_degree(dst_g, ones16, zeros16)
    dis, dinv, g0 = _tc_a(cntp, xp)
    s0p = _sc_narrow_a(src_g, dst_g, g0, zeros16)
    h1, g1 = _tc_b(s0p, xp, dis, dinv, delta, W0, b0)
    s1 = _sc_wide(src_g, src_shift, dst_g, g1.reshape(2 * NACC, 32), zeros32)
    m2, g2 = _tc_c(s1, h1, dis, dinv, W1, b1, W2p)
    s2p = _sc_narrow_b(src_g, dst_g, g2, zeros16)
    ntype, npos = _tc_d(s2p, m2, dis, dinv, b2p, xp, posp)
    return (ntype[:N], npos[:N])


# speedup vs baseline: 23.2370x; 1.1179x over previous
import functools

import jax
import jax.numpy as jnp
from jax import lax
from jax.experimental import pallas as pl
from jax.experimental.pallas import tpu as pltpu
from jax.experimental.pallas import tpu_sc as plsc

N = 50000
NACC = 50048
ROWS_PT = NACC // 16
E = 800000
EPAD = 819200
NG = EPAD // 128
TRASH = N
GPW_NARROW = NG // 32
GPW_WIDE = NG // 16
KD = 10
KDW = 4
WN = 16
BLK = 2176
GRID = NACC // BLK

_MESH = plsc.VectorSubcoreMesh(core_axis_name="c", subcore_axis_name="s")
_CP = pltpu.CompilerParams(use_tc_tiling_on_sc=False)


def _zero_acc(zeros_hbm, acc, s):
    r0 = s * ROWS_PT
    pltpu.sync_copy(zeros_hbm.at[pl.ds(r0, ROWS_PT)], acc.at[pl.ds(r0, ROWS_PT)])


def _flush_acc(acc, out_hbm, c, s):
    r0 = s * ROWS_PT
    pltpu.sync_copy(acc.at[pl.ds(r0, ROWS_PT)], out_hbm.at[c, pl.ds(r0, ROWS_PT)])


@functools.partial(
    pl.kernel,
    out_type=jax.ShapeDtypeStruct((2, NACC, WN), jnp.float32),
    mesh=_MESH,
    compiler_params=_CP,
    scratch_types=[
        pltpu.VMEM_SHARED((NACC, WN), jnp.float32),
        pltpu.VMEM((GPW_NARROW, 128), jnp.int32),
        pltpu.VMEM((128, WN), jnp.float32),
        pltpu.SemaphoreType.DMA,
    ],
)
def _sc_degree(dst_hbm, ones_hbm, zeros_hbm, out_hbm, acc, dst_v, ones_v, sem):
    c = lax.axis_index("c")
    s = lax.axis_index("s")
    wid = c * 16 + s
    _zero_acc(zeros_hbm, acc, s)
    pltpu.sync_copy(ones_hbm, ones_v)
    pltpu.sync_copy(dst_hbm.at[pl.ds(wid * GPW_NARROW, GPW_NARROW)], dst_v)
    plsc.subcore_barrier()

    def body(g, carry):
        pltpu.async_copy(ones_v, acc.at[dst_v.at[g]], sem, add=True)
        return carry

    lax.fori_loop(0, GPW_NARROW, body, 0)

    def drain(g, carry):
        pltpu.make_async_copy(zeros_hbm.at[pl.ds(0, 128)], ones_v, sem).wait()
        return carry

    lax.fori_loop(0, GPW_NARROW, drain, 0)
    plsc.subcore_barrier()
    _flush_acc(acc, out_hbm, c, s)


def _make_narrow():
    @functools.partial(
        pl.kernel,
        out_type=jax.ShapeDtypeStruct((2, NACC, WN), jnp.float32),
        mesh=_MESH,
        compiler_params=_CP,
        scratch_types=[
            pltpu.VMEM_SHARED((NACC, WN), jnp.float32),
            pltpu.VMEM((GPW_NARROW, 128), jnp.int32),
            pltpu.VMEM((GPW_NARROW, 128), jnp.int32),
            pltpu.VMEM((KD, 128, WN), jnp.float32),
            pltpu.SemaphoreType.DMA((KD,)),
        ],
    )
    def kern(src_hbm, dst_hbm, table_hbm, zeros_hbm, out_hbm,
             acc, src_v, dst_v, msg_v, sems):
        c = lax.axis_index("c")
        s = lax.axis_index("s")
        wid = c * 16 + s
        g0 = wid * GPW_NARROW
        _zero_acc(zeros_hbm, acc, s)
        pltpu.sync_copy(src_hbm.at[pl.ds(g0, GPW_NARROW)], src_v)
        pltpu.sync_copy(dst_hbm.at[pl.ds(g0, GPW_NARROW)], dst_v)
        plsc.subcore_barrier()

        for j in range(KD):
            pltpu.async_copy(table_hbm.at[src_v.at[j]], msg_v.at[j],
                             sems.at[j])

        def chunk(cix, carry):
            base = cix * KD
            for j in range(KD):
                pltpu.make_async_copy(zeros_hbm.at[pl.ds(0, 128)],
                                      msg_v.at[j], sems.at[j]).wait()
                pltpu.sync_copy(msg_v.at[j], acc.at[dst_v.at[base + j]],
                                add=True)

                @pl.when(base + KD + j < GPW_NARROW)
                def _():
                    pltpu.async_copy(table_hbm.at[src_v.at[base + KD + j]],
                                     msg_v.at[j], sems.at[j])
            return carry

        lax.fori_loop(0, GPW_NARROW // KD, chunk, 0)
        plsc.subcore_barrier()
        _flush_acc(acc, out_hbm, c, s)

    return kern


_sc_narrow_a = _make_narrow()
_sc_narrow_b = _make_narrow()

ROWS_REF = 20
NREF = GPW_WIDE // ROWS_REF


@functools.partial(
    pl.kernel,
    out_type=jax.ShapeDtypeStruct((2, NACC, 32), jnp.float32),
    mesh=_MESH,
    compiler_params=_CP,
    scratch_types=[
        pltpu.VMEM_SHARED((NACC, 32), jnp.float32),
        pltpu.VMEM((ROWS_REF, 128), jnp.int32),
        pltpu.VMEM((ROWS_REF, 128), jnp.int32),
        pltpu.VMEM((KDW, 128, 32), jnp.float32),
        pltpu.SemaphoreType.DMA((KDW,)),
    ],
)
def _sc_wide(src_hbm, dst_hbm, table_a_hbm, table_b_hbm, zeros_hbm, out_hbm,
             acc, src_v, dst_v, msg_v, sems):
    c = lax.axis_index("c")
    s = lax.axis_index("s")
    _zero_acc(zeros_hbm, acc, s)
    plsc.subcore_barrier()

    def gather(row_ix, j):
        @pl.when(c == 0)
        def _():
            pltpu.async_copy(table_a_hbm.at[src_v.at[row_ix]], msg_v.at[j],
                             sems.at[j])

        @pl.when(c == 1)
        def _():
            pltpu.async_copy(table_b_hbm.at[src_v.at[row_ix]], msg_v.at[j],
                             sems.at[j])

    def refill(r, carry):
        g0 = s * GPW_WIDE + r * ROWS_REF
        pltpu.sync_copy(src_hbm.at[pl.ds(g0, ROWS_REF)], src_v)
        pltpu.sync_copy(dst_hbm.at[pl.ds(g0, ROWS_REF)], dst_v)

        for j in range(KDW):
            gather(j, j)

        def chunk(cix, carry2):
            base = cix * KDW
            for j in range(KDW):
                pltpu.make_async_copy(zeros_hbm.at[pl.ds(0, 128)],
                                      msg_v.at[j], sems.at[j]).wait()
                pltpu.sync_copy(msg_v.at[j], acc.at[dst_v.at[base + j]],
                                add=True)

                @pl.when(base + KDW + j < ROWS_REF)
                def _():
                    gather(base + KDW + j, j)
            return carry2

        lax.fori_loop(0, ROWS_REF // KDW, chunk, 0)
        return carry

    lax.fori_loop(0, NREF, refill, 0)
    plsc.subcore_barrier()
    _flush_acc(acc, out_hbm, c, s)


def _row_spec(width):
    return pl.BlockSpec((BLK, width), lambda i: (i, 0))


def _pair_spec(width):
    return pl.BlockSpec((2, BLK, width), lambda i: (0, i, 0))


def _full_spec(r, cols):
    return pl.BlockSpec((r, cols), lambda i: (0, 0))


def _tc_a_body(cnt_ref, x_ref, g0_ref):
    deg = cnt_ref[0, :, 0:1] + cnt_ref[1, :, 0:1] + 1.0
    dis = lax.rsqrt(deg)
    dinv = 1.0 / deg
    x = x_ref[...]
    pad = jnp.zeros((BLK, WN - 4), jnp.float32)
    g0_ref[...] = jnp.concatenate([dis * x, dis, dinv, x, pad], axis=1)


def _tc_a(cntp, xp):
    return pl.pallas_call(
        _tc_a_body,
        grid=(GRID,),
        in_specs=[_pair_spec(WN), _row_spec(1)],
        out_specs=_row_spec(WN),
        out_shape=jax.ShapeDtypeStruct((NACC, WN), jnp.float32),
    )(cntp, xp)


def _tc_b_body(s0_ref, g0_ref, delta_ref, w0_ref, b0_ref,
               h1_ref, g1a_ref, g1b_ref):
    g0 = g0_ref[...]
    dis = g0[:, 1:2]
    dinv = g0[:, 2:3]
    x = g0[:, 3:4]
    s0 = s0_ref[0] + s0_ref[1]
    a0 = dis * s0[:, 0:1] + dinv * x
    a1 = dis * s0[:, 1:2] + dinv
    w0d = jnp.sum(delta_ref[...].reshape(4, 1) * w0_ref[1:5, :], axis=0,
                  keepdims=True)
    z0 = a0 * w0_ref[0:1, :] + a1 * w0d + b0_ref[...]
    h1 = jnp.maximum(z0, 0.0)
    h1_ref[...] = h1
    g1 = dis * h1
    g1a_ref[...] = g1[:, 0:32]
    g1b_ref[...] = g1[:, 32:64]


def _tc_b(s0p, g0, delta, W0, b0):
    return pl.pallas_call(
        _tc_b_body,
        grid=(GRID,),
        in_specs=[_pair_spec(WN), _row_spec(WN),
                  _full_spec(1, 4), _full_spec(5, 64), _full_spec(1, 64)],
        out_specs=[_row_spec(64), _row_spec(32), _row_spec(32)],
        out_shape=[
            jax.ShapeDtypeStruct((NACC, 64), jnp.float32),
            jax.ShapeDtypeStruct((NACC, 32), jnp.float32),
            jax.ShapeDtypeStruct((NACC, 32), jnp.float32),
        ],
    )(s0p, g0, delta.reshape(1, 4), W0, b0.reshape(1, 64))


def _tc_c_body(s1_ref, h1_ref, g0_ref, w1_ref, b1_ref, w2_ref, g2_ref):
    g0 = g0_ref[...]
    dis = g0[:, 1:2]
    dinv = g0[:, 2:3]
    agg = jnp.concatenate([s1_ref[0], s1_ref[1]], axis=1)
    t = dis * agg + dinv * h1_ref[...]
    z1 = jnp.dot(t, w1_ref[...], preferred_element_type=jnp.float32)
    h2 = jnp.maximum(z1 + b1_ref[...], 0.0)
    m2 = jnp.dot(h2, w2_ref[...], preferred_element_type=jnp.float32)
    pad = jnp.zeros((BLK, WN - 8), jnp.float32)
    g2_ref[...] = jnp.concatenate([dis * m2, m2, pad], axis=1)


def _tc_c(s1, h1, g0, W1, b1, W2p):
    return pl.pallas_call(
        _tc_c_body,
        grid=(GRID,),
        in_specs=[_pair_spec(32), _row_spec(64), _row_spec(WN),
                  _full_spec(64, 64), _full_spec(1, 64), _full_spec(64, 4)],
        out_specs=_row_spec(WN),
        out_shape=jax.ShapeDtypeStruct((NACC, WN), jnp.float32),
    )(s1, h1, g0, W1, b1.reshape(1, 64), W2p)


def _tc_d_body(s2_ref, g2_ref, g0_ref, b2_ref, pos_ref, ntype_ref, npos_ref):
    g0 = g0_ref[...]
    dis = g0[:, 1:2]
    dinv = g0[:, 2:3]
    x = g0[:, 3:4]
    m2 = g2_ref[:, 4:8]
    o = (dis * (s2_ref[0, :, 0:4] + s2_ref[1, :, 0:4])
         + dinv * m2 + b2_ref[...])
    npos_ref[...] = pos_ref[...] + o[:, 0:2]
    ntype_ref[...] = x + o[:, 2:3]


def _tc_d(s2p, g2, g0, b2p, posp):
    return pl.pallas_call(
        _tc_d_body,
        grid=(GRID,),
        in_specs=[_pair_spec(WN), _row_spec(WN), _row_spec(WN),
                  _full_spec(1, 4), _row_spec(2)],
        out_specs=[_row_spec(1), _row_spec(2)],
        out_shape=[
            jax.ShapeDtypeStruct((NACC, 1), jnp.float32),
            jax.ShapeDtypeStruct((NACC, 2), jnp.float32),
        ],
    )(s2p, g2, g0, b2p, posp)


def kernel(x, pos, delta, edge_index, W0, b0, W1, b1, W2, b2):
    src = edge_index[0].astype(jnp.int32)
    dst = edge_index[1].astype(jnp.int32)
    pad = EPAD - E
    src_g = jnp.concatenate([src, jnp.zeros((pad,), jnp.int32)]).reshape(NG, 128)
    dst_g = jnp.concatenate([dst, jnp.full((pad,), TRASH, jnp.int32)]).reshape(NG, 128)
    xp = jnp.pad(x, ((0, NACC - N), (0, 0)))
    posp = jnp.pad(pos, ((0, NACC - N), (0, 0)))
    zeros16 = jnp.zeros((NACC, WN), jnp.float32)
    zeros32 = jnp.zeros((NACC, 32), jnp.float32)
    ones16 = jnp.ones((128, WN), jnp.float32)
    W2p = jnp.pad(W2, ((0, 0), (0, 1)))
    b2p = jnp.pad(b2, (0, 1)).reshape(1, 4)

    cntp = _sc_degree(dst_g, ones16, zeros16)
    g0 = _tc_a(cntp, xp)
    s0p = _sc_narrow_a(src_g, dst_g, g0, zeros16)
    h1, g1a, g1b = _tc_b(s0p, g0, delta, W0, b0)
    s1 = _sc_wide(src_g, dst_g, g1a, g1b, zeros32)
    g2 = _tc_c(s1, h1, g0, W1, b1, W2p)
    s2p = _sc_narrow_b(src_g, dst_g, g2, zeros16)
    ntype, npos = _tc_d(s2p, g2, g0, b2p, posp)
    return (ntype[:N], npos[:N])


# speedup vs baseline: 23.5721x; 1.0144x over previous
import functools

import jax
import jax.numpy as jnp
from jax import lax
from jax.experimental import pallas as pl
from jax.experimental.pallas import tpu as pltpu
from jax.experimental.pallas import tpu_sc as plsc

N = 50000
NACC = 50048
ROWS_PT = NACC // 16
E = 800000
EPAD = 819200
NG = EPAD // 128
TRASH = N
GPW_NARROW = NG // 32
GPW_WIDE = NG // 16
KD = 10
KDW = 4
WN = 16
BLK = 3128
GRID = NACC // BLK

_MESH = plsc.VectorSubcoreMesh(core_axis_name="c", subcore_axis_name="s")
_CP = pltpu.CompilerParams(use_tc_tiling_on_sc=False)


def _zero_acc(zeros_hbm, acc, s):
    r0 = s * ROWS_PT
    pltpu.sync_copy(zeros_hbm.at[pl.ds(r0, ROWS_PT)], acc.at[pl.ds(r0, ROWS_PT)])


def _flush_acc(acc, out_hbm, c, s):
    r0 = s * ROWS_PT
    pltpu.sync_copy(acc.at[pl.ds(r0, ROWS_PT)], out_hbm.at[c, pl.ds(r0, ROWS_PT)])


@functools.partial(
    pl.kernel,
    out_type=jax.ShapeDtypeStruct((2, NACC, WN), jnp.float32),
    mesh=_MESH,
    compiler_params=_CP,
    scratch_types=[
        pltpu.VMEM_SHARED((NACC, WN), jnp.float32),
        pltpu.VMEM((GPW_NARROW, 128), jnp.int32),
        pltpu.VMEM((128, WN), jnp.float32),
        pltpu.SemaphoreType.DMA,
    ],
)
def _sc_degree(dst_hbm, ones_hbm, zeros_hbm, out_hbm, acc, dst_v, ones_v, sem):
    c = lax.axis_index("c")
    s = lax.axis_index("s")
    wid = c * 16 + s
    _zero_acc(zeros_hbm, acc, s)
    pltpu.sync_copy(ones_hbm, ones_v)
    pltpu.sync_copy(dst_hbm.at[pl.ds(wid * GPW_NARROW, GPW_NARROW)], dst_v)
    plsc.subcore_barrier()

    def body(g, carry):
        pltpu.async_copy(ones_v, acc.at[dst_v.at[g]], sem, add=True)
        return carry

    lax.fori_loop(0, GPW_NARROW, body, 0)

    def drain(g, carry):
        pltpu.make_async_copy(zeros_hbm.at[pl.ds(0, 128)], ones_v, sem).wait()
        return carry

    lax.fori_loop(0, GPW_NARROW, drain, 0)
    plsc.subcore_barrier()
    _flush_acc(acc, out_hbm, c, s)


def _make_narrow():
    @functools.partial(
        pl.kernel,
        out_type=jax.ShapeDtypeStruct((2, NACC, WN), jnp.float32),
        mesh=_MESH,
        compiler_params=_CP,
        scratch_types=[
            pltpu.VMEM_SHARED((NACC, WN), jnp.float32),
            pltpu.VMEM((GPW_NARROW, 128), jnp.int32),
            pltpu.VMEM((GPW_NARROW, 128), jnp.int32),
            pltpu.VMEM((KD, 128, WN), jnp.float32),
            pltpu.SemaphoreType.DMA((KD,)),
        ],
    )
    def kern(src_hbm, dst_hbm, table_hbm, zeros_hbm, out_hbm,
             acc, src_v, dst_v, msg_v, sems):
        c = lax.axis_index("c")
        s = lax.axis_index("s")
        wid = c * 16 + s
        g0 = wid * GPW_NARROW
        _zero_acc(zeros_hbm, acc, s)
        pltpu.sync_copy(src_hbm.at[pl.ds(g0, GPW_NARROW)], src_v)
        pltpu.sync_copy(dst_hbm.at[pl.ds(g0, GPW_NARROW)], dst_v)
        plsc.subcore_barrier()

        for j in range(KD):
            pltpu.async_copy(table_hbm.at[src_v.at[j]], msg_v.at[j],
                             sems.at[j])

        def chunk(cix, carry):
            base = cix * KD
            for j in range(KD):
                pltpu.make_async_copy(zeros_hbm.at[pl.ds(0, 128)],
                                      msg_v.at[j], sems.at[j]).wait()
                pltpu.sync_copy(msg_v.at[j], acc.at[dst_v.at[base + j]],
                                add=True)

                @pl.when(base + KD + j < GPW_NARROW)
                def _():
                    pltpu.async_copy(table_hbm.at[src_v.at[base + KD + j]],
                                     msg_v.at[j], sems.at[j])
            return carry

        lax.fori_loop(0, GPW_NARROW // KD, chunk, 0)
        plsc.subcore_barrier()
        _flush_acc(acc, out_hbm, c, s)

    return kern


_sc_narrow_a = _make_narrow()
_sc_narrow_b = _make_narrow()

ROWS_REF = 20
NREF = GPW_WIDE // ROWS_REF


@functools.partial(
    pl.kernel,
    out_type=jax.ShapeDtypeStruct((2, NACC, 32), jnp.float32),
    mesh=_MESH,
    compiler_params=_CP,
    scratch_types=[
        pltpu.VMEM_SHARED((NACC, 32), jnp.float32),
        pltpu.VMEM((ROWS_REF, 128), jnp.int32),
        pltpu.VMEM((ROWS_REF, 128), jnp.int32),
        pltpu.VMEM((KDW, 128, 32), jnp.float32),
        pltpu.SemaphoreType.DMA((KDW,)),
    ],
)
def _sc_wide(src_hbm, dst_hbm, table_a_hbm, table_b_hbm, zeros_hbm, out_hbm,
             acc, src_v, dst_v, msg_v, sems):
    c = lax.axis_index("c")
    s = lax.axis_index("s")
    _zero_acc(zeros_hbm, acc, s)
    plsc.subcore_barrier()

    def gather(row_ix, j):
        @pl.when(c == 0)
        def _():
            pltpu.async_copy(table_a_hbm.at[src_v.at[row_ix]], msg_v.at[j],
                             sems.at[j])

        @pl.when(c == 1)
        def _():
            pltpu.async_copy(table_b_hbm.at[src_v.at[row_ix]], msg_v.at[j],
                             sems.at[j])

    def refill(r, carry):
        g0 = s * GPW_WIDE + r * ROWS_REF
        pltpu.sync_copy(src_hbm.at[pl.ds(g0, ROWS_REF)], src_v)
        pltpu.sync_copy(dst_hbm.at[pl.ds(g0, ROWS_REF)], dst_v)

        for j in range(KDW):
            gather(j, j)

        def chunk(cix, carry2):
            base = cix * KDW
            for j in range(KDW):
                pltpu.make_async_copy(zeros_hbm.at[pl.ds(0, 128)],
                                      msg_v.at[j], sems.at[j]).wait()
                pltpu.sync_copy(msg_v.at[j], acc.at[dst_v.at[base + j]],
                                add=True)

                @pl.when(base + KDW + j < ROWS_REF)
                def _():
                    gather(base + KDW + j, j)
            return carry2

        lax.fori_loop(0, ROWS_REF // KDW, chunk, 0)
        return carry

    lax.fori_loop(0, NREF, refill, 0)
    plsc.subcore_barrier()
    _flush_acc(acc, out_hbm, c, s)


def _row_spec(width):
    return pl.BlockSpec((BLK, width), lambda i: (i, 0))


def _pair_spec(width):
    return pl.BlockSpec((2, BLK, width), lambda i: (0, i, 0))


def _full_spec(r, cols):
    return pl.BlockSpec((r, cols), lambda i: (0, 0))


def _tc_a_body(cnt_ref, x_ref, g0_ref):
    deg = cnt_ref[0, :, 0:1] + cnt_ref[1, :, 0:1] + 1.0
    dis = lax.rsqrt(deg)
    dinv = 1.0 / deg
    x = x_ref[...]
    pad = jnp.zeros((BLK, WN - 4), jnp.float32)
    g0_ref[...] = jnp.concatenate([dis * x, dis, dinv, x, pad], axis=1)


def _tc_a(cntp, xp):
    return pl.pallas_call(
        _tc_a_body,
        grid=(GRID,),
        in_specs=[_pair_spec(WN), _row_spec(1)],
        out_specs=_row_spec(WN),
        out_shape=jax.ShapeDtypeStruct((NACC, WN), jnp.float32),
    )(cntp, xp)


def _tc_b_body(s0_ref, g0_ref, delta_ref, w0_ref, b0_ref,
               h1_ref, g1a_ref, g1b_ref):
    g0 = g0_ref[...]
    dis = g0[:, 1:2]
    dinv = g0[:, 2:3]
    x = g0[:, 3:4]
    s0 = s0_ref[0] + s0_ref[1]
    a0 = dis * s0[:, 0:1] + dinv * x
    a1 = dis * s0[:, 1:2] + dinv
    w0d = jnp.sum(delta_ref[...].reshape(4, 1) * w0_ref[1:5, :], axis=0,
                  keepdims=True)
    z0 = a0 * w0_ref[0:1, :] + a1 * w0d + b0_ref[...]
    h1 = jnp.maximum(z0, 0.0)
    h1_ref[...] = h1
    g1 = dis * h1
    g1a_ref[...] = g1[:, 0:32]
    g1b_ref[...] = g1[:, 32:64]


def _tc_b(s0p, g0, delta, W0, b0):
    return pl.pallas_call(
        _tc_b_body,
        grid=(GRID,),
        in_specs=[_pair_spec(WN), _row_spec(WN),
                  _full_spec(1, 4), _full_spec(5, 64), _full_spec(1, 64)],
        out_specs=[_row_spec(64), _row_spec(32), _row_spec(32)],
        out_shape=[
            jax.ShapeDtypeStruct((NACC, 64), jnp.float32),
            jax.ShapeDtypeStruct((NACC, 32), jnp.float32),
            jax.ShapeDtypeStruct((NACC, 32), jnp.float32),
        ],
    )(s0p, g0, delta.reshape(1, 4), W0, b0.reshape(1, 64))


def _tc_c_body(s1_ref, h1_ref, g0_ref, w1_ref, b1_ref, w2_ref, g2_ref):
    g0 = g0_ref[...]
    dis = g0[:, 1:2]
    dinv = g0[:, 2:3]
    agg = jnp.concatenate([s1_ref[0], s1_ref[1]], axis=1)
    t = dis * agg + dinv * h1_ref[...]
    z1 = jnp.dot(t, w1_ref[...], preferred_element_type=jnp.float32)
    h2 = jnp.maximum(z1 + b1_ref[...], 0.0)
    m2 = jnp.dot(h2, w2_ref[...], preferred_element_type=jnp.float32)
    pad = jnp.zeros((BLK, WN - 8), jnp.float32)
    g2_ref[...] = jnp.concatenate([dis * m2, m2, pad], axis=1)


def _tc_c(s1, h1, g0, W1, b1, W2p):
    return pl.pallas_call(
        _tc_c_body,
        grid=(GRID,),
        in_specs=[_pair_spec(32), _row_spec(64), _row_spec(WN),
                  _full_spec(64, 64), _full_spec(1, 64), _full_spec(64, 4)],
        out_specs=_row_spec(WN),
        out_shape=jax.ShapeDtypeStruct((NACC, WN), jnp.float32),
    )(s1, h1, g0, W1, b1.reshape(1, 64), W2p)


def _tc_d_body(s2_ref, g2_ref, g0_ref, b2_ref, pos_ref, ntype_ref, npos_ref):
    g0 = g0_ref[...]
    dis = g0[:, 1:2]
    dinv = g0[:, 2:3]
    x = g0[:, 3:4]
    m2 = g2_ref[:, 4:8]
    o = (dis * (s2_ref[0, :, 0:4] + s2_ref[1, :, 0:4])
         + dinv * m2 + b2_ref[...])
    npos_ref[...] = pos_ref[...] + o[:, 0:2]
    ntype_ref[...] = x + o[:, 2:3]


def _tc_d(s2p, g2, g0, b2p, posp):
    return pl.pallas_call(
        _tc_d_body,
        grid=(GRID,),
        in_specs=[_pair_spec(WN), _row_spec(WN), _row_spec(WN),
                  _full_spec(1, 4), _row_spec(2)],
        out_specs=[_row_spec(1), _row_spec(2)],
        out_shape=[
            jax.ShapeDtypeStruct((N, 1), jnp.float32),
            jax.ShapeDtypeStruct((N, 2), jnp.float32),
        ],
    )(s2p, g2, g0, b2p, posp)


def kernel(x, pos, delta, edge_index, W0, b0, W1, b1, W2, b2):
    src = edge_index[0].astype(jnp.int32)
    dst = edge_index[1].astype(jnp.int32)
    pad = EPAD - E
    src_g = jnp.concatenate([src, jnp.zeros((pad,), jnp.int32)]).reshape(NG, 128)
    dst_g = jnp.concatenate([dst, jnp.full((pad,), TRASH, jnp.int32)]).reshape(NG, 128)
    zeros16 = jnp.zeros((NACC, WN), jnp.float32)
    zeros32 = jnp.zeros((NACC, 32), jnp.float32)
    ones16 = jnp.ones((128, WN), jnp.float32)
    W2p = jnp.pad(W2, ((0, 0), (0, 1)))
    b2p = jnp.pad(b2, (0, 1)).reshape(1, 4)

    cntp = _sc_degree(dst_g, ones16, zeros16)
    g0 = _tc_a(cntp, x)
    s0p = _sc_narrow_a(src_g, dst_g, g0, zeros16)
    h1, g1a, g1b = _tc_b(s0p, g0, delta, W0, b0)
    s1 = _sc_wide(src_g, dst_g, g1a, g1b, zeros32)
    g2 = _tc_c(s1, h1, g0, W1, b1, W2p)
    s2p = _sc_narrow_b(src_g, dst_g, g2, zeros16)
    ntype, npos = _tc_d(s2p, g2, g0, b2p, pos)
    return (ntype, npos)
